# baseline (speedup 1.0000x reference)
import functools

import jax
import jax.numpy as jnp
from jax import lax
from jax.experimental import pallas as pl
from jax.experimental.pallas import tpu as pltpu
from jax.experimental.pallas import tpu_sc as plsc

N = 10000
E = 320000
F_IN = 128
N_HID = 16
N_CLS = 40

NC = 2
NS = 16
NW = NC * NS
CK = 128
CTOT = E // CK
NCHW = 80
CPAD = NW * NCHW
R_PAD = 10240
RPS = R_PAD // NS
NBUF = 8
PF = 5

_mesh = plsc.VectorSubcoreMesh(core_axis_name="c", subcore_axis_name="s")
_sc_params = pltpu.CompilerParams(use_tc_tiling_on_sc=False)


def _deg_body(e4_hbm, zer_hbm, one_hbm, out_hbm, col_v, ones_v, acc, sem):
    c = lax.axis_index("c")
    s = lax.axis_index("s")
    wid = s * NC + c
    base = wid * NCHW
    sl = pl.ds(s * RPS, RPS)

    pltpu.async_copy(zer_hbm.at[sl], acc.at[sl], sem)
    pltpu.async_copy(e4_hbm.at[1, pl.ds(base, NCHW)], col_v, sem)
    pltpu.async_copy(one_hbm, ones_v, sem)
    pltpu.make_async_copy(zer_hbm.at[sl], acc.at[sl], sem).wait()
    pltpu.make_async_copy(e4_hbm.at[1, pl.ds(base, NCHW)], col_v, sem).wait()
    pltpu.make_async_copy(one_hbm, ones_v, sem).wait()
    plsc.subcore_barrier()

    @pl.loop(0, NCHW, step=20)
    def _(j):
        for b in range(20):
            pltpu.async_copy(ones_v, acc.at[col_v.at[j + b]], sem, add=True)
        for b in range(20):
            pltpu.make_async_copy(ones_v, acc.at[col_v.at[j + b]], sem).wait()

    plsc.subcore_barrier()
    pltpu.sync_copy(acc.at[sl], out_hbm.at[c].at[sl])


def _scatter_body(D, g_hbm, zer_hbm, e4_hbm, out_hbm,
                  idx_v, bufs, g_st, acc, gsem, ssem):
    c = lax.axis_index("c")
    s = lax.axis_index("s")
    wid = s * NC + c
    base = wid * NCHW
    sl = pl.ds(s * RPS, RPS)

    pltpu.async_copy(g_hbm.at[sl], g_st.at[sl], ssem.at[0])

    @pl.when(c == 0)
    def _():
        pltpu.async_copy(g_hbm.at[sl], acc.at[sl], ssem.at[1])

    @pl.when(c != 0)
    def _():
        pltpu.async_copy(zer_hbm.at[sl], acc.at[sl], ssem.at[1])

    pltpu.async_copy(e4_hbm.at[:, pl.ds(base, NCHW)], idx_v, ssem.at[2])

    pltpu.make_async_copy(g_hbm.at[sl], g_st.at[sl], ssem.at[0]).wait()
    pltpu.make_async_copy(zer_hbm.at[sl], acc.at[sl], ssem.at[1]).wait()
    pltpu.make_async_copy(e4_hbm.at[:, pl.ds(base, NCHW)], idx_v,
                          ssem.at[2]).wait()
    plsc.subcore_barrier()

    def start_gather(j, b):
        pltpu.async_copy(g_st.at[idx_v.at[0, j]], bufs.at[b], gsem.at[b])

    def wait_gather(j, b):
        pltpu.make_async_copy(g_st.at[idx_v.at[0, j]], bufs.at[b],
                              gsem.at[b]).wait()

    def start_scatter(j, b):
        pltpu.async_copy(bufs.at[b], acc.at[idx_v.at[1, j]], ssem.at[b],
                         add=True)

    def wait_scatter(j, b):
        pltpu.make_async_copy(bufs.at[b], acc.at[idx_v.at[1, j]],
                              ssem.at[b]).wait()

    for p in range(PF):
        start_gather(p, p)

    @pl.loop(0, NCHW, step=NBUF)
    def _(j0):
        for i in range(NBUF):
            j = j0 + i
            bn = (i + PF) % NBUF
            jn = j + PF

            @pl.when(jn < NCHW)
            def _():
                @pl.when(jn >= NBUF)
                def _():
                    wait_scatter(jn - NBUF, bn)
                start_gather(jn, bn)

            wait_gather(j, i)
            start_scatter(j, i)

    for b in range(NBUF):
        wait_scatter(0, b)

    plsc.subcore_barrier()
    pltpu.sync_copy(acc.at[sl], out_hbm.at[c].at[sl])


def _deg_call(e4, zer1, one1):
    return pl.kernel(
        _deg_body,
        out_type=jax.ShapeDtypeStruct((NC, R_PAD, 1), jnp.float32),
        mesh=_mesh,
        scratch_types=[
            pltpu.VMEM((NCHW, CK), jnp.int32),
            pltpu.VMEM((CK, 1), jnp.float32),
            pltpu.VMEM_SHARED((R_PAD, 1), jnp.float32),
            pltpu.SemaphoreType.DMA,
        ],
        compiler_params=_sc_params,
    )(e4, zer1, one1)


def _scatter_call(D, g, zer, e4):
    return pl.kernel(
        functools.partial(_scatter_body, D),
        out_type=jax.ShapeDtypeStruct((NC, R_PAD, D), jnp.float32),
        mesh=_mesh,
        scratch_types=[
            pltpu.VMEM((2, NCHW, CK), jnp.int32),
            pltpu.VMEM((NBUF, CK, D), jnp.float32),
            pltpu.VMEM_SHARED((R_PAD, D), jnp.float32),
            pltpu.VMEM_SHARED((R_PAD, D), jnp.float32),
            pltpu.SemaphoreType.DMA((NBUF,)),
            pltpu.SemaphoreType.DMA((NBUF,)),
        ],
        compiler_params=_sc_params,
    )(g, zer, e4)


def _mm1_body(x_ref, w_ref, h_ref):
    h_ref[...] = jnp.dot(x_ref[...], w_ref[...],
                         preferred_element_type=jnp.float32)


def _scale_body(degp_ref, h_ref, g_ref, d_ref):
    degp = degp_ref[...]
    deg = degp[0] + degp[1] + 1.0
    d = lax.rsqrt(deg)
    d_ref[...] = d
    g_ref[...] = h_ref[...] * d[:N, :]


def _mid_body(a_ref, d_ref, b1_ref, w2_ref, g2_ref):
    a = a_ref[...]
    d = d_ref[...]
    o1 = jnp.maximum((a[0] + a[1]) * d + b1_ref[...], 0.0)
    h2 = jnp.dot(o1, w2_ref[...], preferred_element_type=jnp.float32)
    g2_ref[...] = h2 * d


def _fin_body(a_ref, d_ref, b2_ref, o_ref):
    a = a_ref[...]
    o = (a[0, :N] + a[1, :N]) * d_ref[...][:N] + b2_ref[...]
    m = jnp.max(o, axis=1, keepdims=True)
    e = jnp.exp(o - m)
    lse = jnp.log(jnp.sum(e, axis=1, keepdims=True))
    o_ref[...] = o - m - lse


def _mm1(x, W1):
    return pl.pallas_call(
        _mm1_body,
        grid=(1,),
        in_specs=[pl.BlockSpec((N, F_IN), lambda i: (0, 0)),
                  pl.BlockSpec((F_IN, N_HID), lambda i: (0, 0))],
        out_specs=pl.BlockSpec((N, N_HID), lambda i: (0, 0)),
        out_shape=jax.ShapeDtypeStruct((N, N_HID), jnp.float32),
    )(x, W1)


def _scale(degp, h):
    return pl.pallas_call(
        _scale_body,
        grid=(1,),
        in_specs=[pl.BlockSpec((NC, R_PAD, 1), lambda i: (0, 0, 0)),
                  pl.BlockSpec((N, N_HID), lambda i: (0, 0))],
        out_specs=[pl.BlockSpec((N, N_HID), lambda i: (0, 0)),
                   pl.BlockSpec((R_PAD, 1), lambda i: (0, 0))],
        out_shape=[jax.ShapeDtypeStruct((R_PAD, N_HID), jnp.float32),
                   jax.ShapeDtypeStruct((R_PAD, 1), jnp.float32)],
    )(degp, h)


def _mid(a1, d, b1r, W2):
    return pl.pallas_call(
        _mid_body,
        grid=(1,),
        in_specs=[pl.BlockSpec((NC, R_PAD, N_HID), lambda i: (0, 0, 0)),
                  pl.BlockSpec((R_PAD, 1), lambda i: (0, 0)),
                  pl.BlockSpec((1, N_HID), lambda i: (0, 0)),
                  pl.BlockSpec((N_HID, N_CLS), lambda i: (0, 0))],
        out_specs=pl.BlockSpec((R_PAD, N_CLS), lambda i: (0, 0)),
        out_shape=jax.ShapeDtypeStruct((R_PAD, N_CLS), jnp.float32),
    )(a1, d, b1r, W2)


def _fin(a2, d, b2r):
    return pl.pallas_call(
        _fin_body,
        grid=(1,),
        in_specs=[pl.BlockSpec((NC, R_PAD, N_CLS), lambda i: (0, 0, 0)),
                  pl.BlockSpec((R_PAD, 1), lambda i: (0, 0)),
                  pl.BlockSpec((1, N_CLS), lambda i: (0, 0))],
        out_specs=pl.BlockSpec((N, N_CLS), lambda i: (0, 0)),
        out_shape=jax.ShapeDtypeStruct((N, N_CLS), jnp.float32),
    )(a2, d, b2r)


def kernel(x, edge_index, W1, b1, W2, b2):
    e4 = jnp.pad(edge_index.reshape(2, CTOT, CK),
                 ((0, 0), (0, CPAD - CTOT), (0, 0)), constant_values=N)

    zer1 = jnp.zeros((R_PAD, 1), jnp.float32)
    one1 = jnp.ones((CK, 1), jnp.float32)
    zer16 = jnp.zeros((R_PAD, N_HID), jnp.float32)
    zer40 = jnp.zeros((R_PAD, N_CLS), jnp.float32)

    degp = _deg_call(e4, zer1, one1)
    h = _mm1(x, W1)
    g1, d = _scale(degp, h)
    a1 = _scatter_call(N_HID, g1, zer16, e4)
    g2 = _mid(a1, d, b1.reshape(1, N_HID), W2)
    a2 = _scatter_call(N_CLS, g2, zer40, e4)
    return _fin(a2, d, b2.reshape(1, N_CLS))
